# baseline (speedup 1.0000x reference)
import functools

import jax
import jax.numpy as jnp
from jax import lax
from jax.experimental import pallas as pl
from jax.experimental.pallas import tpu as pltpu
from jax.experimental.pallas import tpu_sc as plsc

N = 50000
NP = 50176
E = 800000
K = 112
EP = 802816
NCHUNK = EP // K
KS = 128
EPS = 823296
NCHUNKS = EPS // KS
NS = 16
NC = 2
RPT = NP // NS
CPB = 6
CPB_H = 14
DUMP = N
R = 512
G = NP // R


def _mesh():
    return plsc.VectorSubcoreMesh(core_axis_name="c", subcore_axis_name="s")


def _sc_hist(colL2d, colG2d, ones_hbm, zeros16):
    @functools.partial(
        pl.kernel, mesh=_mesh(),
        compiler_params=pltpu.CompilerParams(use_tc_tiling_on_sc=False),
        out_type=[jax.ShapeDtypeStruct((NP, 16), jnp.float32),
                  jax.ShapeDtypeStruct((NP, 16), jnp.float32)],
        scratch_types=[
            pltpu.VMEM_SHARED((NP, 16), jnp.float32),
            pltpu.VMEM((2, CPB_H, K), jnp.int32),
            pltpu.VMEM((K, 16), jnp.float32),
            pltpu.SemaphoreType.DMA,
        ],
    )
    def k(colL_h, colG_h, ones_h, z_h, hL_h, hG_h, acc, colv, onesv, sem):
        c = lax.axis_index("c")
        s = lax.axis_index("s")
        pltpu.sync_copy(z_h, acc.at[pl.ds(s * RPT, RPT)])
        pltpu.sync_copy(ones_h, onesv)
        plsc.subcore_barrier()
        per_tile = NCHUNK // NS
        nblocks = per_tile // CPB_H
        tbase = s * per_tile

        def run(col_h, out_h):
            pltpu.sync_copy(col_h.at[pl.ds(tbase, CPB_H)], colv.at[0])

            def body(b, carry):
                m = lax.rem(b, 2)
                mn = lax.rem(b + 1, 2)

                @pl.when(b > 0)
                def _():
                    for j in range(CPB_H):
                        pltpu.make_async_copy(
                            onesv, acc.at[colv.at[mn, j]], sem).wait()

                @pl.when(b + 1 < nblocks)
                def _():
                    pltpu.sync_copy(
                        col_h.at[pl.ds(tbase + (b + 1) * CPB_H, CPB_H)],
                        colv.at[mn])

                for j in range(CPB_H):
                    pltpu.async_copy(
                        onesv, acc.at[colv.at[m, j]], sem, add=True)
                return carry

            lax.fori_loop(0, nblocks, body, 0)
            last = (nblocks - 1) % 2
            for j in range(CPB_H):
                pltpu.make_async_copy(
                    onesv, acc.at[colv.at[last, j]], sem).wait()
            plsc.subcore_barrier()
            pltpu.sync_copy(acc.at[pl.ds(s * RPT, RPT)],
                            out_h.at[pl.ds(s * RPT, RPT)])

        @pl.when(c == 0)
        def _():
            run(colL_h, hL_h)

        @pl.when(c == 1)
        def _():
            run(colG_h, hG_h)

    return k(colL2d, colG2d, ones_hbm, zeros16)


def _sc_scatter(row2d, col2d, yA, yB, zeros32):
    @functools.partial(
        pl.kernel, mesh=_mesh(),
        compiler_params=pltpu.CompilerParams(use_tc_tiling_on_sc=False),
        out_type=[jax.ShapeDtypeStruct((NP, 32), jnp.float32),
                  jax.ShapeDtypeStruct((NP, 32), jnp.float32)],
        scratch_types=[
            pltpu.VMEM_SHARED((NP, 32), jnp.float32),
            pltpu.VMEM((2, CPB, KS), jnp.int32),
            pltpu.VMEM((2, CPB, KS), jnp.int32),
            pltpu.VMEM((CPB, KS, 32), jnp.float32),
            pltpu.SemaphoreType.DMA,
            pltpu.SemaphoreType.DMA,
        ],
    )
    def k(row_h, col_h, yA_h, yB_h, z_h, s0_h, s1_h,
          acc, rowv, colv, gbuf, sem_g, sem_s):
        c = lax.axis_index("c")
        s = lax.axis_index("s")
        pltpu.sync_copy(z_h, acc.at[pl.ds(s * RPT, RPT)])
        plsc.subcore_barrier()
        per_tile = NCHUNKS // NS
        nblocks = per_tile // CPB

        def run(y_h, out_h):
            tbase = s * per_tile
            pltpu.sync_copy(row_h.at[pl.ds(tbase, CPB)], rowv.at[0])
            pltpu.sync_copy(col_h.at[pl.ds(tbase, CPB)], colv.at[0])
            for j in range(CPB):
                pltpu.async_copy(y_h.at[rowv.at[0, j]], gbuf.at[j], sem_g)

            def body(b, carry):
                m = lax.rem(b, 2)
                mn = lax.rem(b + 1, 2)

                @pl.when(b + 1 < nblocks)
                def _():
                    nb = tbase + (b + 1) * CPB
                    pltpu.sync_copy(row_h.at[pl.ds(nb, CPB)], rowv.at[mn])
                    pltpu.sync_copy(col_h.at[pl.ds(nb, CPB)], colv.at[mn])

                sd = []
                for j in range(CPB):
                    pltpu.make_async_copy(
                        y_h.at[rowv.at[m, j]], gbuf.at[j], sem_g).wait()
                    sd.append(pltpu.async_copy(
                        gbuf.at[j], acc.at[colv.at[m, j]], sem_s, add=True))

                @pl.when(b + 1 < nblocks)
                def _():
                    for j in range(CPB):
                        sd[j].wait()
                        pltpu.async_copy(
                            y_h.at[rowv.at[mn, j]], gbuf.at[j], sem_g)

                @pl.when(b + 1 >= nblocks)
                def _():
                    for j in range(CPB):
                        sd[j].wait()
                return carry

            lax.fori_loop(0, nblocks, body, 0)
            plsc.subcore_barrier()
            pltpu.sync_copy(acc.at[pl.ds(s * RPT, RPT)],
                            out_h.at[pl.ds(s * RPT, RPT)])

        @pl.when(c == 0)
        def _():
            run(yA_h, s0_h)

        @pl.when(c == 1)
        def _():
            run(yB_h, s1_h)

    return k(row2d, col2d, yA, yB, zeros32)


def _tc_first(xp, h, W):
    F = xp.shape[1]

    def body(x_ref, h_ref, w_ref, ya_ref, yb_ref, hc_ref):
        j = pl.program_id(0)
        dinv = lax.rsqrt(h_ref[:, 0:1] + 1.0)
        xb = x_ref[...]
        y = dinv * jnp.dot(xb, w_ref[...], preferred_element_type=jnp.float32)
        ya_ref[...] = y[:, :32]
        yb_ref[...] = y[:, 32:]
        cs = jnp.sum(xb, axis=0, keepdims=True)
        tot = jnp.sum(xb).reshape(1, 1)
        vec = jnp.concatenate(
            [cs, jnp.zeros((1, 127 - F), jnp.float32), tot], axis=1)

        @pl.when(j == 0)
        def _():
            hc_ref[...] = jnp.zeros((8, 128), jnp.float32)

        hc_ref[...] += jnp.broadcast_to(vec, (8, 128))

    return pl.pallas_call(
        body,
        grid=(G,),
        in_specs=[pl.BlockSpec((R, F), lambda j: (j, 0)),
                  pl.BlockSpec((R, 16), lambda j: (j, 0)),
                  pl.BlockSpec((F, 64), lambda j: (0, 0))],
        out_specs=[pl.BlockSpec((R, 32), lambda j: (j, 0)),
                   pl.BlockSpec((R, 32), lambda j: (j, 0)),
                   pl.BlockSpec((8, 128), lambda j: (0, 0))],
        out_shape=[jax.ShapeDtypeStruct((NP, 32), jnp.float32),
                   jax.ShapeDtypeStruct((NP, 32), jnp.float32),
                   jax.ShapeDtypeStruct((8, 128), jnp.float32)],
    )(xp, h, W)


def _tc_mid(S0, S1, yA, yB, h, W, b):
    def body(s0, s1, ya, yb, hr, wr, br, oa, ob):
        dinv = lax.rsqrt(hr[:, 0:1] + 1.0)
        agg = jnp.concatenate([s0[...] + ya[...], s1[...] + yb[...]], axis=1)
        h = jnp.maximum(dinv * agg + br[...], 0.0)
        y = dinv * jnp.dot(h, wr[...], preferred_element_type=jnp.float32)
        oa[...] = y[:, :32]
        ob[...] = y[:, 32:]

    blk = lambda w: pl.BlockSpec((R, w), lambda j: (j, 0))
    return pl.pallas_call(
        body,
        grid=(G,),
        in_specs=[blk(32), blk(32), blk(32), blk(32), blk(16),
                  pl.BlockSpec((64, 64), lambda j: (0, 0)),
                  pl.BlockSpec((1, 64), lambda j: (0, 0))],
        out_specs=[blk(32), blk(32)],
        out_shape=[jax.ShapeDtypeStruct((NP, 32), jnp.float32),
                   jax.ShapeDtypeStruct((NP, 32), jnp.float32)],
    )(S0, S1, yA, yB, h, W, b)


def _tc_last(S0, S1, yA, yB, h, b):
    def body(s0, s1, ya, yb, hr, br, pool_ref):
        j = pl.program_id(0)
        dinv = lax.rsqrt(hr[:, 0:1] + 1.0)
        agg = jnp.concatenate([s0[...] + ya[...], s1[...] + yb[...]], axis=1)
        h = jnp.maximum(dinv * agg + br[...], 0.0)
        rows = j * R + lax.broadcasted_iota(jnp.int32, (R, 1), 0)
        h = jnp.where(rows < N, h, 0.0)
        m = jnp.max(h, axis=0, keepdims=True)

        @pl.when(j == 0)
        def _():
            pool_ref[...] = jnp.zeros((8, 64), jnp.float32)

        pool_ref[...] = jnp.maximum(pool_ref[...], jnp.broadcast_to(m, (8, 64)))

    blk = lambda w: pl.BlockSpec((R, w), lambda j: (j, 0))
    return pl.pallas_call(
        body,
        grid=(G,),
        in_specs=[blk(32), blk(32), blk(32), blk(32), blk(16),
                  pl.BlockSpec((1, 64), lambda j: (0, 0))],
        out_specs=pl.BlockSpec((8, 64), lambda j: (0, 0)),
        out_shape=jax.ShapeDtypeStruct((8, 64), jnp.float32),
    )(S0, S1, yA, yB, h, b)


def _tc_mlp(v8, p1w, p1b, p2wt):
    def body(vr, w1, b1, w2t, o):
        h = jnp.maximum(
            jnp.dot(vr[...], w1[...], preferred_element_type=jnp.float32)
            + b1[...], 0.0)
        o[...] = jnp.broadcast_to(
            jnp.sum(h * w2t[...], axis=1, keepdims=True), (8, 128))

    return pl.pallas_call(
        body,
        out_shape=jax.ShapeDtypeStruct((8, 128), jnp.float32),
    )(v8, p1w, p1b, p2wt)


def _pad_edges(edge_index):
    row = edge_index[0]
    col = edge_index[1]
    rowp = jnp.concatenate(
        [row, jnp.zeros((EPS - E,), jnp.int32)]).reshape(NCHUNKS, KS)
    colp = jnp.concatenate(
        [col, jnp.full((EPS - E,), DUMP, jnp.int32)]).reshape(NCHUNKS, KS)
    colh = jnp.concatenate(
        [col, jnp.full((EP - E,), DUMP, jnp.int32)]).reshape(NCHUNK, K)
    return rowp, colp, colh


def _gcn_module(x, rowp, colp, h, Ws, bs):
    xp = jnp.pad(x, ((0, NP - N), (0, 0)))
    zeros32 = jnp.zeros((RPT, 32), jnp.float32)

    yA, yB, hc = _tc_first(xp, h, Ws[0])
    for li in (1, 2):
        S0, S1 = _sc_scatter(rowp, colp, yA, yB, zeros32)
        yA, yB = _tc_mid(S0, S1, yA, yB, h,
                         Ws[li], bs[li - 1].reshape(1, 64))
    S0, S1 = _sc_scatter(rowp, colp, yA, yB, zeros32)
    pooled = _tc_last(S0, S1, yA, yB, h, bs[2].reshape(1, 64))
    return pooled, hc


def kernel(lifted_x, grounded_x, lifted_edge_index, grounded_edge_index,
           lifted_batch, grounded_batch, lW1, lb1, lW2, lb2, lW3, lb3,
           gW1, gb1, gW2, gb2, gW3, gb3, p1W, p1b, p2W, p2b):
    rowL, colL, colhL = _pad_edges(lifted_edge_index)
    rowG, colG, colhG = _pad_edges(grounded_edge_index)
    zeros16 = jnp.zeros((RPT, 16), jnp.float32)
    ones = jnp.ones((K, 16), jnp.float32)
    hL, hG = _sc_hist(colhL, colhG, ones, zeros16)
    poolL, hcL = _gcn_module(lifted_x, rowL, colL, hL,
                             (lW1, lW2, lW3), (lb1, lb2, lb3))
    poolG, hcG = _gcn_module(grounded_x, rowG, colG, hG,
                             (gW1, gW2, gW3), (gb1, gb2, gb3))
    hl = hcL[0, :16] / hcL[0, 127]
    hg = hcG[0, :7] / hcG[0, 127]
    v = jnp.concatenate(
        [poolL[0], hl, poolG[0], hg, jnp.zeros((1,), jnp.float32)])
    v8 = jnp.broadcast_to(v[None, :], (8, 152))
    p1Wp = jnp.pad(p1W, ((0, 1), (0, 0)))
    o = _tc_mlp(v8, p1Wp, p1b.reshape(1, 128), p2W.reshape(1, 128))
    return o[0, 0] + p2b


# speedup vs baseline: 1.9412x; 1.9412x over previous
import functools

import jax
import jax.numpy as jnp
from jax import lax
from jax.experimental import pallas as pl
from jax.experimental.pallas import tpu as pltpu
from jax.experimental.pallas import tpu_sc as plsc

N = 50000
NP = 50176
E = 800000
K = 112
EP = 802816
NCHUNK = EP // K
NS = 16
NC = 2
RPT = NP // NS
CPB = 7
CPB_H = 14
DUMP = N
R = 512
G = NP // R


def _mesh():
    return plsc.VectorSubcoreMesh(core_axis_name="c", subcore_axis_name="s")


def _sc_hist(colL2d, colG2d, ones_hbm, zeros16):
    @functools.partial(
        pl.kernel, mesh=_mesh(),
        compiler_params=pltpu.CompilerParams(use_tc_tiling_on_sc=False),
        out_type=[jax.ShapeDtypeStruct((NP, 16), jnp.float32),
                  jax.ShapeDtypeStruct((NP, 16), jnp.float32)],
        scratch_types=[
            pltpu.VMEM_SHARED((NP, 16), jnp.float32),
            pltpu.VMEM((2, CPB_H, K), jnp.int32),
            pltpu.VMEM((K, 16), jnp.float32),
            pltpu.SemaphoreType.DMA,
        ],
    )
    def k(colL_h, colG_h, ones_h, z_h, hL_h, hG_h, acc, colv, onesv, sem):
        c = lax.axis_index("c")
        s = lax.axis_index("s")
        pltpu.sync_copy(z_h, acc.at[pl.ds(s * RPT, RPT)])
        pltpu.sync_copy(ones_h, onesv)
        plsc.subcore_barrier()
        per_tile = NCHUNK // NS
        nblocks = per_tile // CPB_H
        tbase = s * per_tile

        def run(col_h, out_h):
            pltpu.sync_copy(col_h.at[pl.ds(tbase, CPB_H)], colv.at[0])

            def body(b, carry):
                m = lax.rem(b, 2)
                mn = lax.rem(b + 1, 2)

                @pl.when(b > 0)
                def _():
                    for j in range(CPB_H):
                        pltpu.make_async_copy(
                            onesv, acc.at[colv.at[mn, j]], sem).wait()

                @pl.when(b + 1 < nblocks)
                def _():
                    pltpu.sync_copy(
                        col_h.at[pl.ds(tbase + (b + 1) * CPB_H, CPB_H)],
                        colv.at[mn])

                for j in range(CPB_H):
                    pltpu.async_copy(
                        onesv, acc.at[colv.at[m, j]], sem, add=True)
                return carry

            lax.fori_loop(0, nblocks, body, 0)
            last = (nblocks - 1) % 2
            for j in range(CPB_H):
                pltpu.make_async_copy(
                    onesv, acc.at[colv.at[last, j]], sem).wait()
            plsc.subcore_barrier()
            pltpu.sync_copy(acc.at[pl.ds(s * RPT, RPT)],
                            out_h.at[pl.ds(s * RPT, RPT)])

        @pl.when(c == 0)
        def _():
            run(colL_h, hL_h)

        @pl.when(c == 1)
        def _():
            run(colG_h, hG_h)

    return k(colL2d, colG2d, ones_hbm, zeros16)


def _sc_scatter(row2d, col2d, yA, yB, zeros32):
    @functools.partial(
        pl.kernel, mesh=_mesh(),
        compiler_params=pltpu.CompilerParams(use_tc_tiling_on_sc=False),
        out_type=[jax.ShapeDtypeStruct((NP, 32), jnp.float32),
                  jax.ShapeDtypeStruct((NP, 32), jnp.float32)],
        scratch_types=[
            pltpu.VMEM_SHARED((NP, 32), jnp.float32),
            pltpu.VMEM((2, CPB, K), jnp.int32),
            pltpu.VMEM((2, CPB, K), jnp.int32),
            pltpu.VMEM((CPB, K, 32), jnp.float32),
            pltpu.SemaphoreType.DMA,
            pltpu.SemaphoreType.DMA,
        ],
    )
    def k(row_h, col_h, yA_h, yB_h, z_h, s0_h, s1_h,
          acc, rowv, colv, gbuf, sem_g, sem_s):
        c = lax.axis_index("c")
        s = lax.axis_index("s")
        pltpu.sync_copy(z_h, acc.at[pl.ds(s * RPT, RPT)])
        plsc.subcore_barrier()
        per_tile = NCHUNK // NS
        nblocks = per_tile // CPB

        def run(y_h, out_h):
            tbase = s * per_tile
            pltpu.sync_copy(row_h.at[pl.ds(tbase, CPB)], rowv.at[0])
            pltpu.sync_copy(col_h.at[pl.ds(tbase, CPB)], colv.at[0])
            for j in range(CPB):
                pltpu.async_copy(y_h.at[rowv.at[0, j]], gbuf.at[j], sem_g)

            def body(b, carry):
                m = lax.rem(b, 2)
                mn = lax.rem(b + 1, 2)

                @pl.when(b + 1 < nblocks)
                def _():
                    nb = tbase + (b + 1) * CPB
                    pltpu.sync_copy(row_h.at[pl.ds(nb, CPB)], rowv.at[mn])
                    pltpu.sync_copy(col_h.at[pl.ds(nb, CPB)], colv.at[mn])

                sd = []
                for j in range(CPB):
                    pltpu.make_async_copy(
                        y_h.at[rowv.at[m, j]], gbuf.at[j], sem_g).wait()
                    sd.append(pltpu.async_copy(
                        gbuf.at[j], acc.at[colv.at[m, j]], sem_s, add=True))

                @pl.when(b + 1 < nblocks)
                def _():
                    for j in range(CPB):
                        sd[j].wait()
                        pltpu.async_copy(
                            y_h.at[rowv.at[mn, j]], gbuf.at[j], sem_g)

                @pl.when(b + 1 >= nblocks)
                def _():
                    for j in range(CPB):
                        sd[j].wait()
                return carry

            lax.fori_loop(0, nblocks, body, 0)
            plsc.subcore_barrier()
            pltpu.sync_copy(acc.at[pl.ds(s * RPT, RPT)],
                            out_h.at[pl.ds(s * RPT, RPT)])

        @pl.when(c == 0)
        def _():
            run(yA_h, s0_h)

        @pl.when(c == 1)
        def _():
            run(yB_h, s1_h)

    return k(row2d, col2d, yA, yB, zeros32)


def _tc_first(xp, h, W):
    F = xp.shape[1]

    def body(x_ref, h_ref, w_ref, ya_ref, yb_ref, hc_ref):
        j = pl.program_id(0)
        dinv = lax.rsqrt(h_ref[:, 0:1] + 1.0)
        xb = x_ref[...]
        y = dinv * jnp.dot(xb, w_ref[...], preferred_element_type=jnp.float32)
        ya_ref[...] = y[:, :32]
        yb_ref[...] = y[:, 32:]
        cs = jnp.sum(xb, axis=0, keepdims=True)
        tot = jnp.sum(xb).reshape(1, 1)
        vec = jnp.concatenate(
            [cs, jnp.zeros((1, 127 - F), jnp.float32), tot], axis=1)

        @pl.when(j == 0)
        def _():
            hc_ref[...] = jnp.zeros((8, 128), jnp.float32)

        hc_ref[...] += jnp.broadcast_to(vec, (8, 128))

    return pl.pallas_call(
        body,
        grid=(G,),
        in_specs=[pl.BlockSpec((R, F), lambda j: (j, 0)),
                  pl.BlockSpec((R, 16), lambda j: (j, 0)),
                  pl.BlockSpec((F, 64), lambda j: (0, 0))],
        out_specs=[pl.BlockSpec((R, 32), lambda j: (j, 0)),
                   pl.BlockSpec((R, 32), lambda j: (j, 0)),
                   pl.BlockSpec((8, 128), lambda j: (0, 0))],
        out_shape=[jax.ShapeDtypeStruct((NP, 32), jnp.float32),
                   jax.ShapeDtypeStruct((NP, 32), jnp.float32),
                   jax.ShapeDtypeStruct((8, 128), jnp.float32)],
    )(xp, h, W)


def _tc_mid(S0, S1, yA, yB, h, W, b):
    def body(s0, s1, ya, yb, hr, wr, br, oa, ob):
        dinv = lax.rsqrt(hr[:, 0:1] + 1.0)
        agg = jnp.concatenate([s0[...] + ya[...], s1[...] + yb[...]], axis=1)
        h = jnp.maximum(dinv * agg + br[...], 0.0)
        y = dinv * jnp.dot(h, wr[...], preferred_element_type=jnp.float32)
        oa[...] = y[:, :32]
        ob[...] = y[:, 32:]

    blk = lambda w: pl.BlockSpec((R, w), lambda j: (j, 0))
    return pl.pallas_call(
        body,
        grid=(G,),
        in_specs=[blk(32), blk(32), blk(32), blk(32), blk(16),
                  pl.BlockSpec((64, 64), lambda j: (0, 0)),
                  pl.BlockSpec((1, 64), lambda j: (0, 0))],
        out_specs=[blk(32), blk(32)],
        out_shape=[jax.ShapeDtypeStruct((NP, 32), jnp.float32),
                   jax.ShapeDtypeStruct((NP, 32), jnp.float32)],
    )(S0, S1, yA, yB, h, W, b)


def _tc_last(S0, S1, yA, yB, h, b):
    def body(s0, s1, ya, yb, hr, br, pool_ref):
        j = pl.program_id(0)
        dinv = lax.rsqrt(hr[:, 0:1] + 1.0)
        agg = jnp.concatenate([s0[...] + ya[...], s1[...] + yb[...]], axis=1)
        h = jnp.maximum(dinv * agg + br[...], 0.0)
        rows = j * R + lax.broadcasted_iota(jnp.int32, (R, 1), 0)
        h = jnp.where(rows < N, h, 0.0)
        m = jnp.max(h, axis=0, keepdims=True)

        @pl.when(j == 0)
        def _():
            pool_ref[...] = jnp.zeros((8, 64), jnp.float32)

        pool_ref[...] = jnp.maximum(pool_ref[...], jnp.broadcast_to(m, (8, 64)))

    blk = lambda w: pl.BlockSpec((R, w), lambda j: (j, 0))
    return pl.pallas_call(
        body,
        grid=(G,),
        in_specs=[blk(32), blk(32), blk(32), blk(32), blk(16),
                  pl.BlockSpec((1, 64), lambda j: (0, 0))],
        out_specs=pl.BlockSpec((8, 64), lambda j: (0, 0)),
        out_shape=jax.ShapeDtypeStruct((8, 64), jnp.float32),
    )(S0, S1, yA, yB, h, b)


def _tc_mlp(v8, p1w, p1b, p2wt):
    def body(vr, w1, b1, w2t, o):
        h = jnp.maximum(
            jnp.dot(vr[...], w1[...], preferred_element_type=jnp.float32)
            + b1[...], 0.0)
        o[...] = jnp.broadcast_to(
            jnp.sum(h * w2t[...], axis=1, keepdims=True), (8, 128))

    return pl.pallas_call(
        body,
        out_shape=jax.ShapeDtypeStruct((8, 128), jnp.float32),
    )(v8, p1w, p1b, p2wt)


def _pad_edges(edge_index):
    row = edge_index[0]
    col = edge_index[1]
    npad = EP - E
    rowp = jnp.concatenate(
        [row, jnp.zeros((npad,), jnp.int32)]).reshape(NCHUNK, K)
    colp = jnp.concatenate(
        [col, jnp.full((npad,), DUMP, jnp.int32)]).reshape(NCHUNK, K)
    return rowp, colp


def _gcn_module(x, rowp, colp, h, Ws, bs):
    xp = jnp.pad(x, ((0, NP - N), (0, 0)))
    zeros32 = jnp.zeros((RPT, 32), jnp.float32)

    yA, yB, hc = _tc_first(xp, h, Ws[0])
    for li in (1, 2):
        S0, S1 = _sc_scatter(rowp, colp, yA, yB, zeros32)
        yA, yB = _tc_mid(S0, S1, yA, yB, h,
                         Ws[li], bs[li - 1].reshape(1, 64))
    S0, S1 = _sc_scatter(rowp, colp, yA, yB, zeros32)
    pooled = _tc_last(S0, S1, yA, yB, h, bs[2].reshape(1, 64))
    return pooled, hc


def kernel(lifted_x, grounded_x, lifted_edge_index, grounded_edge_index,
           lifted_batch, grounded_batch, lW1, lb1, lW2, lb2, lW3, lb3,
           gW1, gb1, gW2, gb2, gW3, gb3, p1W, p1b, p2W, p2b):
    rowL, colL = _pad_edges(lifted_edge_index)
    rowG, colG = _pad_edges(grounded_edge_index)
    zeros16 = jnp.zeros((RPT, 16), jnp.float32)
    ones = jnp.ones((K, 16), jnp.float32)
    hL, hG = _sc_hist(colL, colG, ones, zeros16)
    poolL, hcL = _gcn_module(lifted_x, rowL, colL, hL,
                             (lW1, lW2, lW3), (lb1, lb2, lb3))
    poolG, hcG = _gcn_module(grounded_x, rowG, colG, hG,
                             (gW1, gW2, gW3), (gb1, gb2, gb3))
    hl = hcL[0, :16] / hcL[0, 127]
    hg = hcG[0, :7] / hcG[0, 127]
    v = jnp.concatenate(
        [poolL[0], hl, poolG[0], hg, jnp.zeros((1,), jnp.float32)])
    v8 = jnp.broadcast_to(v[None, :], (8, 152))
    p1Wp = jnp.pad(p1W, ((0, 1), (0, 0)))
    o = _tc_mlp(v8, p1Wp, p1b.reshape(1, 128), p2W.reshape(1, 128))
    return o[0, 0] + p2b
